# baseline (speedup 1.0000x reference)
import functools

import jax
import jax.numpy as jnp
from jax import lax
from jax.experimental import pallas as pl
from jax.experimental.pallas import tpu as pltpu
from jax.experimental.pallas import tpu_sc as plsc

NC = 2
NS = 16
NW = NC * NS
CHUNK = 128
PAIR = 2
NBUF = 3


def _make_gather(V, D, B):
    assert B % (NW * CHUNK * PAIR) == 0
    bpw = B // NW
    npr = bpw // (CHUNK * PAIR)
    assert npr >= NBUF
    ncyc = -(-npr // NBUF)
    mesh = plsc.VectorSubcoreMesh(
        core_axis_name="c", subcore_axis_name="s",
        num_cores=NC, num_subcores=NS)

    @functools.partial(
        pl.kernel,
        out_type=jax.ShapeDtypeStruct((B, D), jnp.float32),
        mesh=mesh,
        scratch_types=[
            pltpu.VMEM((npr * PAIR, CHUNK), jnp.int32),
            [pltpu.VMEM((PAIR * CHUNK, D), jnp.float32)] * NBUF,
            [pltpu.SemaphoreType.DMA] * NBUF,
            [pltpu.SemaphoreType.DMA] * NBUF,
        ],
    )
    def gather_kernel(table_hbm, idx_hbm, out_hbm, idx_v, bufs, gsems, ssems):
        wid = lax.axis_index("s") * NC + lax.axis_index("c")
        base = wid * bpw
        pltpu.sync_copy(idx_hbm.at[wid], idx_v)

        def start_pair(q, b):
            for h in range(PAIR):
                pltpu.async_copy(
                    table_hbm.at[idx_v.at[q * PAIR + h]],
                    bufs[b].at[pl.ds(h * CHUNK, CHUNK)], gsems[b])

        def wait_pair(q, b):
            for h in range(PAIR):
                pltpu.make_async_copy(
                    table_hbm.at[idx_v.at[q * PAIR + h]],
                    bufs[b].at[pl.ds(h * CHUNK, CHUNK)], gsems[b]).wait()

        def out_slice(q):
            return out_hbm.at[pl.ds(base + q * PAIR * CHUNK, PAIR * CHUNK)]

        for b in range(NBUF):
            start_pair(b, b)

        @pl.loop(0, ncyc)
        def _(g):
            q0 = g * NBUF
            for b in range(NBUF):
                @pl.when(q0 + b < npr)
                def _():
                    wait_pair(q0 + b, b)
                    pltpu.async_copy(bufs[b], out_slice(q0 + b), ssems[b])
            for b in range(NBUF):
                @pl.when(q0 + b + NBUF < npr)
                def _():
                    pltpu.make_async_copy(
                        bufs[b], out_slice(q0 + b), ssems[b]).wait()
                    start_pair(q0 + b + NBUF, b)

        for b in range(NBUF):
            last_q = (npr - 1 - b) // NBUF * NBUF + b
            pltpu.make_async_copy(
                bufs[b], out_slice(last_q), ssems[b]).wait()

    return gather_kernel


def kernel(x, table):
    V, D = table.shape
    S, T = x.shape
    B = x.size
    idx = x.T.reshape(NW, B // (NW * CHUNK), CHUNK).astype(jnp.int32)
    out = _make_gather(V, D, B)(table, idx)
    return out.reshape(T, S, D).transpose(1, 0, 2)
